# Initial kernel scaffold; baseline (speedup 1.0000x reference)
import jax
import jax.numpy as jnp
from jax.experimental import pallas as pl


def kernel(grammian):
    raise NotImplementedError("write your pallas kernel here")


# speedup vs baseline: 5.2470x; 5.2470x over previous
import functools

import jax
import jax.numpy as jnp
from jax import lax
from jax.experimental import pallas as pl
from jax.experimental.pallas import tpu as pltpu
from jax.experimental.pallas import tpu_sc as plsc

N = 2048
NW = 32
ROWS_PER_W = N // NW
CHUNK_R = 4
BIG_I32 = 2 ** 30


def _scan_body(g_hbm, gflat_hbm, didx_hbm,
               val_hbm, idx_hbm, gam_hbm,
               didx_v, diag_v, rowbuf, val_v, idx_v, gam_v, sem):
    cid = lax.axis_index("c")
    sid = lax.axis_index("s")
    wid = sid * 2 + cid

    pltpu.sync_copy(didx_hbm, didx_v)
    for j in range(16):
        pltpu.async_copy(gflat_hbm.at[didx_v.at[j]],
                         diag_v.at[pl.ds(j * 128, 128)], sem).wait()

    lane = lax.iota(jnp.int32, 16)
    row0 = wid * ROWS_PER_W

    bv = jnp.full((16,), jnp.inf, jnp.float32)
    bi = jnp.full((16,), BIG_I32, jnp.int32)
    bg = jnp.zeros((16,), jnp.float32)
    dchunk = None
    for t in range(ROWS_PER_W // CHUNK_R):
        i0 = row0 + t * CHUNK_R
        pltpu.sync_copy(g_hbm.at[pl.ds(i0, CHUNK_R)], rowbuf)
        for r in range(CHUNK_R):
            ridx = t * CHUNK_R + r
            i = row0 + ridx
            if ridx % 16 == 0:
                dchunk = diag_v[pl.ds(row0 + ridx, 16)]
            a16 = jnp.full((16,), dchunk[ridx % 16], jnp.float32)

            def v_body(v, c2, r=r, i=i, a16=a16):
                bv, bi, bg = c2
                c16 = rowbuf[r, pl.ds(v * 16, 16)]
                b16 = diag_v[pl.ds(v * 16, 16)]
                j16 = v * 16 + lane
                clt_b = c16 < b16
                clt_a = c16 < a16
                gam = (b16 - c16) / (a16 + b16 - 2.0 * c16 + 1e-8)
                gam = jnp.where(clt_b, gam, 0.0)
                gam = jnp.where(clt_a, gam, 1.0)
                cost = b16 + gam * (c16 - b16)
                cost = jnp.where(clt_b, cost, b16)
                cost = jnp.where(clt_a, cost, a16)
                cost = jnp.where(j16 > i, cost, jnp.inf)
                lin = i * N + j16
                better = cost < bv
                bv = jnp.where(better, cost, bv)
                bi = jnp.where(better, lin, bi)
                bg = jnp.where(better, gam, bg)
                return bv, bi, bg

            bv, bi, bg = lax.fori_loop(0, N // 16, v_body, (bv, bi, bg))

    val_v[...] = bv
    idx_v[...] = bi
    gam_v[...] = bg
    pltpu.sync_copy(val_v, val_hbm.at[wid])
    pltpu.sync_copy(idx_v, idx_hbm.at[wid])
    pltpu.sync_copy(gam_v, gam_hbm.at[wid])


_scan = functools.partial(
    pl.kernel,
    out_type=(jax.ShapeDtypeStruct((NW, 16), jnp.float32),
              jax.ShapeDtypeStruct((NW, 16), jnp.int32),
              jax.ShapeDtypeStruct((NW, 16), jnp.float32)),
    mesh=plsc.VectorSubcoreMesh(core_axis_name="c", subcore_axis_name="s"),
    scratch_types=[
        pltpu.VMEM((16, 128), jnp.int32),
        pltpu.VMEM((N,), jnp.float32),
        pltpu.VMEM((CHUNK_R, N), jnp.float32),
        pltpu.VMEM((16,), jnp.float32),
        pltpu.VMEM((16,), jnp.int32),
        pltpu.VMEM((16,), jnp.float32),
        pltpu.SemaphoreType.DMA,
    ],
)(_scan_body)


def _merge_body(val_ref, idx_ref, gam_ref, out_ref):
    v = val_ref[...]
    ix = idx_ref[...]
    g = gam_ref[...]
    m = jnp.min(v)
    win = jnp.min(jnp.where(v == m, ix, BIG_I32))
    gw = jnp.min(jnp.where(ix == win, g, jnp.inf))
    i_min = win >> 11
    j_min = win & (N - 1)
    flat = (lax.broadcasted_iota(jnp.int32, (16, 128), 0) * 128
            + lax.broadcasted_iota(jnp.int32, (16, 128), 1))
    out_ref[...] = (jnp.where(flat == i_min, gw, 0.0)
                    + jnp.where(flat == j_min, 1.0 - gw, 0.0))


_merge = pl.pallas_call(
    _merge_body,
    out_shape=jax.ShapeDtypeStruct((16, 128), jnp.float32),
)


def kernel(grammian):
    gflat = grammian.reshape(-1)
    didx = (jnp.arange(N, dtype=jnp.int32) * (N + 1)).reshape(16, 128)
    val, ix, gam = _scan(grammian, gflat, didx)
    return _merge(val, ix, gam).reshape(N)


# speedup vs baseline: 6.5835x; 1.2547x over previous
import functools

import jax
import jax.numpy as jnp
from jax import lax
from jax.experimental import pallas as pl
from jax.experimental.pallas import tpu as pltpu
from jax.experimental.pallas import tpu_sc as plsc

N = 2048
NW = 32
ROWS_PER_W = N // NW
BIG_I32 = 2 ** 30


def _scan_body(g_hbm, gflat_hbm, didx_hbm, aidx_hbm,
               val_hbm, idx_hbm, gam_hbm,
               didx_v, diag_v, aidx_v, adiag_v, rowbuf,
               val_v, idx_v, gam_v, dsem, rsem0, rsem1):
    cid = lax.axis_index("c")
    sid = lax.axis_index("s")
    wid = sid * 2 + cid

    pltpu.sync_copy(didx_hbm, didx_v)
    for j in range(16):
        pltpu.async_copy(gflat_hbm.at[didx_v.at[j]],
                         diag_v.at[pl.ds(j * 128, 128)], dsem).wait()
    pltpu.sync_copy(aidx_hbm.at[wid], aidx_v)
    pltpu.async_copy(gflat_hbm.at[aidx_v], adiag_v, dsem).wait()

    lane = lax.iota(jnp.int32, 16)
    rsems = (rsem0, rsem1)

    def start_row(r, buf):
        c0 = 512 * (r // 16)
        i = wid + NW * r
        return pltpu.async_copy(g_hbm.at[i, pl.ds(c0, N - c0)],
                                rowbuf.at[buf, pl.ds(c0, N - c0)],
                                rsems[buf])

    bv = jnp.full((16,), jnp.inf, jnp.float32)
    bi = jnp.full((16,), BIG_I32, jnp.int32)
    bg = jnp.zeros((16,), jnp.float32)

    descs = [start_row(0, 0), None]
    dchunk = None
    for r in range(ROWS_PER_W):
        buf = r % 2
        if r + 1 < ROWS_PER_W:
            descs[1 - buf] = start_row(r + 1, 1 - buf)
        descs[buf].wait()
        i = wid + NW * r
        if r % 16 == 0:
            dchunk = adiag_v[pl.ds(r, 16)]
        a16 = jnp.full((16,), dchunk[r % 16], jnp.float32)
        ibase = i * N

        def v_body(v, c2, buf=buf, a16=a16, i=i, ibase=ibase):
            bv, bi, bg = c2
            c16 = rowbuf[buf, pl.ds(v * 16, 16)]
            b16 = diag_v[pl.ds(v * 16, 16)]
            j16 = v * 16 + lane
            clt_b = c16 < b16
            clt_a = c16 < a16
            gam = (b16 - c16) / (a16 + b16 - 2.0 * c16 + 1e-8)
            gam = jnp.where(clt_b, gam, 0.0)
            gam = jnp.where(clt_a, gam, 1.0)
            cost = b16 + gam * (c16 - b16)
            cost = jnp.where(clt_b, cost, b16)
            cost = jnp.where(clt_a, cost, a16)
            cost = jnp.where(j16 > i, cost, jnp.inf)
            lin = ibase + j16
            better = cost < bv
            bv = jnp.where(better, cost, bv)
            bi = jnp.where(better, lin, bi)
            bg = jnp.where(better, gam, bg)
            return bv, bi, bg

        bv, bi, bg = lax.fori_loop(32 * (r // 16), N // 16, v_body,
                                   (bv, bi, bg), unroll=2)

    val_v[...] = bv
    idx_v[...] = bi
    gam_v[...] = bg
    pltpu.sync_copy(val_v, val_hbm.at[wid])
    pltpu.sync_copy(idx_v, idx_hbm.at[wid])
    pltpu.sync_copy(gam_v, gam_hbm.at[wid])


_scan = functools.partial(
    pl.kernel,
    out_type=(jax.ShapeDtypeStruct((NW, 16), jnp.float32),
              jax.ShapeDtypeStruct((NW, 16), jnp.int32),
              jax.ShapeDtypeStruct((NW, 16), jnp.float32)),
    mesh=plsc.VectorSubcoreMesh(core_axis_name="c", subcore_axis_name="s"),
    scratch_types=[
        pltpu.VMEM((16, 128), jnp.int32),
        pltpu.VMEM((N,), jnp.float32),
        pltpu.VMEM((ROWS_PER_W,), jnp.int32),
        pltpu.VMEM((ROWS_PER_W,), jnp.float32),
        pltpu.VMEM((2, N), jnp.float32),
        pltpu.VMEM((16,), jnp.float32),
        pltpu.VMEM((16,), jnp.int32),
        pltpu.VMEM((16,), jnp.float32),
        pltpu.SemaphoreType.DMA,
        pltpu.SemaphoreType.DMA,
        pltpu.SemaphoreType.DMA,
    ],
)(_scan_body)


def _merge_body(val_ref, idx_ref, gam_ref, out_ref):
    v = val_ref[...]
    ix = idx_ref[...]
    g = gam_ref[...]
    m = jnp.min(v)
    win = jnp.min(jnp.where(v == m, ix, BIG_I32))
    gw = jnp.min(jnp.where(ix == win, g, jnp.inf))
    i_min = win >> 11
    j_min = win & (N - 1)
    flat = (lax.broadcasted_iota(jnp.int32, (16, 128), 0) * 128
            + lax.broadcasted_iota(jnp.int32, (16, 128), 1))
    out_ref[...] = (jnp.where(flat == i_min, gw, 0.0)
                    + jnp.where(flat == j_min, 1.0 - gw, 0.0))


_merge = pl.pallas_call(
    _merge_body,
    out_shape=jax.ShapeDtypeStruct((16, 128), jnp.float32),
)


def kernel(grammian):
    gflat = grammian.reshape(-1)
    didx = (jnp.arange(N, dtype=jnp.int32) * (N + 1)).reshape(16, 128)
    aidx = ((jnp.arange(NW, dtype=jnp.int32)[:, None]
             + NW * jnp.arange(ROWS_PER_W, dtype=jnp.int32)[None, :])
            * (N + 1))
    val, ix, gam = _scan(grammian, gflat, didx, aidx)
    return _merge(val, ix, gam).reshape(N)


# speedup vs baseline: 7.1105x; 1.0801x over previous
import functools

import jax
import jax.numpy as jnp
from jax import lax
from jax.experimental import pallas as pl
from jax.experimental.pallas import tpu as pltpu
from jax.experimental.pallas import tpu_sc as plsc

N = 2048
NW = 32
ROWS_PER_W = N // NW
BIG_I32 = 2 ** 30


def _scan_body(gflat_hbm, didx_hbm, aidx_hbm,
               val_hbm, idx_hbm, gam_hbm,
               didx_v, diag_v, aidx_v, adiag_v, rowbuf,
               val_v, idx_v, gam_v, dsem, rsem0, rsem1):
    cid = lax.axis_index("c")
    sid = lax.axis_index("s")
    wid = sid * 2 + cid

    lane = lax.iota(jnp.int32, 16)
    rsems = (rsem0, rsem1)

    def start_row(r, buf):
        c0 = 512 * (r // 16)
        i = wid + NW * r
        return pltpu.async_copy(gflat_hbm.at[pl.ds(i * N + c0, N - c0)],
                                rowbuf.at[buf, pl.ds(c0, N - c0)],
                                rsems[buf])

    pltpu.sync_copy(didx_hbm, didx_v)
    pltpu.sync_copy(aidx_hbm.at[wid], aidx_v)
    dgather = [pltpu.async_copy(gflat_hbm.at[didx_v.at[j]],
                                diag_v.at[pl.ds(j * 128, 128)], dsem)
               for j in range(16)]
    dgather.append(pltpu.async_copy(gflat_hbm.at[aidx_v], adiag_v, dsem))
    row0_desc = start_row(0, 0)
    for d in dgather:
        d.wait()

    bv = jnp.full((16,), jnp.inf, jnp.float32)
    bi = jnp.full((16,), BIG_I32, jnp.int32)
    bg = jnp.zeros((16,), jnp.float32)

    descs = [row0_desc, None]
    dchunk = None
    for r in range(ROWS_PER_W):
        buf = r % 2
        if r + 1 < ROWS_PER_W:
            descs[1 - buf] = start_row(r + 1, 1 - buf)
        descs[buf].wait()
        i = wid + NW * r
        if r % 16 == 0:
            dchunk = adiag_v[pl.ds(r, 16)]
        a16 = jnp.full((16,), dchunk[r % 16], jnp.float32)
        ibase = i * N

        def v_body(v, c2, buf=buf, a16=a16, i=i, ibase=ibase):
            bv, bi, bg = c2
            c16 = rowbuf[buf, pl.ds(v * 16, 16)]
            b16 = diag_v[pl.ds(v * 16, 16)]
            j16 = v * 16 + lane
            clt_b = c16 < b16
            clt_a = c16 < a16
            gam = (b16 - c16) / (a16 + b16 - 2.0 * c16 + 1e-8)
            gam = jnp.where(clt_b, gam, 0.0)
            gam = jnp.where(clt_a, gam, 1.0)
            cost = b16 + gam * (c16 - b16)
            cost = jnp.where(clt_b, cost, b16)
            cost = jnp.where(clt_a, cost, a16)
            cost = jnp.where(j16 > i, cost, jnp.inf)
            lin = ibase + j16
            better = cost < bv
            bv = jnp.where(better, cost, bv)
            bi = jnp.where(better, lin, bi)
            bg = jnp.where(better, gam, bg)
            return bv, bi, bg

        bv, bi, bg = lax.fori_loop(32 * (r // 16), N // 16, v_body,
                                   (bv, bi, bg), unroll=4)

    val_v[...] = bv
    idx_v[...] = bi
    gam_v[...] = bg
    pltpu.sync_copy(val_v, val_hbm.at[wid])
    pltpu.sync_copy(idx_v, idx_hbm.at[wid])
    pltpu.sync_copy(gam_v, gam_hbm.at[wid])


_scan = functools.partial(
    pl.kernel,
    out_type=(jax.ShapeDtypeStruct((NW, 16), jnp.float32),
              jax.ShapeDtypeStruct((NW, 16), jnp.int32),
              jax.ShapeDtypeStruct((NW, 16), jnp.float32)),
    mesh=plsc.VectorSubcoreMesh(core_axis_name="c", subcore_axis_name="s"),
    scratch_types=[
        pltpu.VMEM((16, 128), jnp.int32),
        pltpu.VMEM((N,), jnp.float32),
        pltpu.VMEM((ROWS_PER_W,), jnp.int32),
        pltpu.VMEM((ROWS_PER_W,), jnp.float32),
        pltpu.VMEM((2, N), jnp.float32),
        pltpu.VMEM((16,), jnp.float32),
        pltpu.VMEM((16,), jnp.int32),
        pltpu.VMEM((16,), jnp.float32),
        pltpu.SemaphoreType.DMA,
        pltpu.SemaphoreType.DMA,
        pltpu.SemaphoreType.DMA,
    ],
)(_scan_body)


def _merge_body(val_ref, idx_ref, gam_ref, out_ref):
    v = val_ref[...]
    ix = idx_ref[...]
    g = gam_ref[...]
    m = jnp.min(v)
    win = jnp.min(jnp.where(v == m, ix, BIG_I32))
    gw = jnp.min(jnp.where(ix == win, g, jnp.inf))
    i_min = win >> 11
    j_min = win & (N - 1)
    flat = (lax.broadcasted_iota(jnp.int32, (16, 128), 0) * 128
            + lax.broadcasted_iota(jnp.int32, (16, 128), 1))
    out_ref[...] = (jnp.where(flat == i_min, gw, 0.0)
                    + jnp.where(flat == j_min, 1.0 - gw, 0.0))


_merge = pl.pallas_call(
    _merge_body,
    out_shape=jax.ShapeDtypeStruct((16, 128), jnp.float32),
)


def kernel(grammian):
    gflat = grammian.reshape(-1)
    didx = (jnp.arange(N, dtype=jnp.int32) * (N + 1)).reshape(16, 128)
    aidx = ((jnp.arange(NW, dtype=jnp.int32)[:, None]
             + NW * jnp.arange(ROWS_PER_W, dtype=jnp.int32)[None, :])
            * (N + 1))
    val, ix, gam = _scan(gflat, didx, aidx)
    return _merge(val, ix, gam).reshape(N)


# speedup vs baseline: 10.4956x; 1.4761x over previous
import functools

import jax
import jax.numpy as jnp
from jax import lax
from jax.experimental import pallas as pl
from jax.experimental.pallas import tpu as pltpu
from jax.experimental.pallas import tpu_sc as plsc

N = 2048
NW = 32
BANDS_PER_W = 8
BIG_I32 = 2 ** 30


def _diag_body(g_ref, out_ref):
    blk = g_ref[...]
    eye = (lax.broadcasted_iota(jnp.int32, (128, 128), 0)
           == lax.broadcasted_iota(jnp.int32, (128, 128), 1))
    out_ref[0, 0, :] = jnp.sum(jnp.where(eye, blk, 0.0), axis=1)


_diag = pl.pallas_call(
    _diag_body,
    grid=(16,),
    in_specs=[pl.BlockSpec((128, 128), lambda k: (k, k))],
    out_specs=pl.BlockSpec((1, 1, 128), lambda k: (k, 0, 0)),
    out_shape=jax.ShapeDtypeStruct((16, 1, 128), jnp.float32),
)


def _scan_body(g_hbm, ddiag_hbm, cand_hbm, candi_hbm,
               ddiag_v, rowbuf, stage, stagei, rsem0, rsem1):
    cid = lax.axis_index("c")
    sid = lax.axis_index("s")
    wid = sid * 2 + cid

    lane = lax.iota(jnp.int32, 16)
    rsems = (rsem0, rsem1)

    def start_band(u, buf):
        row0 = pl.multiple_of(8 * (wid + NW * u), 8)
        c0 = 256 * u
        return pltpu.async_copy(
            g_hbm.at[pl.ds(row0, 8), pl.ds(c0, N - c0)],
            rowbuf.at[buf, slice(None), pl.ds(c0, N - c0)],
            rsems[buf])

    band0_desc = start_band(0, 0)
    pltpu.sync_copy(ddiag_hbm, ddiag_v)

    bv = jnp.full((16,), jnp.inf, jnp.float32)
    bi = jnp.full((16,), BIG_I32, jnp.int32)
    bg = jnp.zeros((16,), jnp.float32)

    w_even = (wid & 1) == 0
    descs = [band0_desc, None]
    for u in range(BANDS_PER_W):
        buf = u % 2
        if u + 1 < BANDS_PER_W:
            descs[1 - buf] = start_band(u + 1, 1 - buf)
        descs[buf].wait()
        b = wid + NW * u
        for r in range(8):
            i = 8 * b + r
            chunk = ddiag_v[pl.ds((i >> 4) * 16, 16)]
            a_s = jnp.where(w_even, chunk[r], chunk[(r + 8) % 16])
            a16 = jnp.full((16,), a_s, jnp.float32)
            ibase = i * N

            def v_body(v, c2, buf=buf, a16=a16, i=i, ibase=ibase, r=r):
                bv, bi, bg = c2
                c16 = rowbuf[buf, r, pl.ds(v * 16, 16)]
                b16 = ddiag_v[pl.ds(v * 16, 16)]
                j16 = v * 16 + lane
                clt_b = c16 < b16
                clt_a = c16 < a16
                gam = (b16 - c16) / (a16 + b16 - 2.0 * c16 + 1e-8)
                gam = jnp.where(clt_b, gam, 0.0)
                gam = jnp.where(clt_a, gam, 1.0)
                cost = b16 + gam * (c16 - b16)
                cost = jnp.where(clt_b, cost, b16)
                cost = jnp.where(clt_a, cost, a16)
                cost = jnp.where(j16 > i, cost, jnp.inf)
                lin = ibase + j16
                better = cost < bv
                bv = jnp.where(better, cost, bv)
                bi = jnp.where(better, lin, bi)
                bg = jnp.where(better, gam, bg)
                return bv, bi, bg

            bv, bi, bg = lax.fori_loop(16 * u, N // 16, v_body,
                                       (bv, bi, bg), unroll=4)

    stage[0, pl.ds(0, 16)] = bv
    stage[1, pl.ds(0, 16)] = bg
    stagei[0, pl.ds(0, 16)] = bi
    pltpu.sync_copy(stage, cand_hbm.at[wid])
    pltpu.sync_copy(stagei, candi_hbm.at[wid])


_scan = functools.partial(
    pl.kernel,
    out_type=(jax.ShapeDtypeStruct((NW, 8, 128), jnp.float32),
              jax.ShapeDtypeStruct((NW, 8, 128), jnp.int32)),
    mesh=plsc.VectorSubcoreMesh(core_axis_name="c", subcore_axis_name="s"),
    compiler_params=pltpu.CompilerParams(use_tc_tiling_on_sc=True),
    scratch_types=[
        pltpu.VMEM((N,), jnp.float32),
        pltpu.VMEM((2, 8, N), jnp.float32),
        pltpu.VMEM((8, 128), jnp.float32),
        pltpu.VMEM((8, 128), jnp.int32),
        pltpu.SemaphoreType.DMA,
        pltpu.SemaphoreType.DMA,
    ],
)(_scan_body)


def _merge_body(cand_ref, candi_ref, out_ref):
    v = cand_ref[:, 0, :16]
    g = cand_ref[:, 1, :16]
    ix = candi_ref[:, 0, :16]
    m = jnp.min(v)
    win = jnp.min(jnp.where(v == m, ix, BIG_I32))
    gw = jnp.min(jnp.where(ix == win, g, jnp.inf))
    i_min = win >> 11
    j_min = win & (N - 1)
    flat = (lax.broadcasted_iota(jnp.int32, (16, 128), 0) * 128
            + lax.broadcasted_iota(jnp.int32, (16, 128), 1))
    out_ref[...] = (jnp.where(flat == i_min, gw, 0.0)
                    + jnp.where(flat == j_min, 1.0 - gw, 0.0))


_merge = pl.pallas_call(
    _merge_body,
    out_shape=jax.ShapeDtypeStruct((16, 128), jnp.float32),
)


def kernel(grammian):
    ddiag = _diag(grammian).reshape(N)
    cand, candi = _scan(grammian, ddiag)
    return _merge(cand, candi).reshape(N)


# speedup vs baseline: 10.9547x; 1.0437x over previous
import functools

import jax
import jax.numpy as jnp
from jax import lax
from jax.experimental import pallas as pl
from jax.experimental.pallas import tpu as pltpu
from jax.experimental.pallas import tpu_sc as plsc

N = 2048
NW = 32
BANDS_PER_W = 8
BIG_I32 = 2 ** 30


def _scan_body(g_hbm, cand_hbm, candi_hbm,
               ddiag_v, rowbuf, stage, stagei, dtiles, dloc, spdiag,
               rsem0, rsem1, dtsem):
    cid = lax.axis_index("c")
    sid = lax.axis_index("s")
    wid = sid * 2 + cid

    lane = lax.iota(jnp.int32, 16)
    rsems = (rsem0, rsem1)

    def start_band(u, buf):
        row0 = pl.multiple_of(8 * (wid + NW * u), 8)
        c0 = 256 * u
        return pltpu.async_copy(
            g_hbm.at[pl.ds(row0, 8), pl.ds(c0, N - c0)],
            rowbuf.at[buf, slice(None), pl.ds(c0, N - c0)],
            rsems[buf])

    band0_desc = start_band(0, 0)

    col0 = pl.multiple_of(128 * sid, 128)
    dt_descs = []
    for tt in range(16):
        row_off = pl.multiple_of(128 * sid + 8 * tt, 8)
        dt_descs.append(pltpu.async_copy(
            g_hbm.at[pl.ds(row_off, 8), pl.ds(col0, 128)],
            dtiles.at[tt], dtsem))
    for d in dt_descs:
        d.wait()
    for k in range(8):
        acc = jnp.zeros((16,), jnp.float32)
        for e in range(16):
            eidx = 16 * k + e
            tt, r = eidx // 8, eidx % 8
            c = 8 * tt + r
            chunk = dtiles[tt, r, pl.ds((c // 16) * 16, 16)]
            acc = jnp.where(lane == e, chunk[c % 16], acc)
        dloc[pl.ds(16 * k, 16)] = acc
    pltpu.sync_copy(dloc, spdiag.at[pl.ds(128 * sid, 128)])
    plsc.subcore_barrier()
    pltpu.sync_copy(spdiag, ddiag_v)

    bv = jnp.full((16,), jnp.inf, jnp.float32)
    bi = jnp.full((16,), BIG_I32, jnp.int32)
    bg = jnp.zeros((16,), jnp.float32)

    w_even = (wid & 1) == 0
    descs = [band0_desc, None]
    for u in range(BANDS_PER_W):
        buf = u % 2
        if u + 1 < BANDS_PER_W:
            descs[1 - buf] = start_band(u + 1, 1 - buf)
        descs[buf].wait()
        b = wid + NW * u
        for r in range(8):
            i = 8 * b + r
            chunk = ddiag_v[pl.ds((i >> 4) * 16, 16)]
            a_s = jnp.where(w_even, chunk[r], chunk[(r + 8) % 16])
            a16 = jnp.full((16,), a_s, jnp.float32)
            ibase = i * N

            def v_body(v, c2, buf=buf, a16=a16, i=i, ibase=ibase, r=r):
                bv, bi, bg = c2
                c16 = rowbuf[buf, r, pl.ds(v * 16, 16)]
                b16 = ddiag_v[pl.ds(v * 16, 16)]
                j16 = v * 16 + lane
                clt_b = c16 < b16
                clt_a = c16 < a16
                gam = (b16 - c16) / (a16 + b16 - 2.0 * c16 + 1e-8)
                gam = jnp.where(clt_b, gam, 0.0)
                gam = jnp.where(clt_a, gam, 1.0)
                cost = b16 + gam * (c16 - b16)
                cost = jnp.where(clt_b, cost, b16)
                cost = jnp.where(clt_a, cost, a16)
                cost = jnp.where(j16 > i, cost, jnp.inf)
                lin = ibase + j16
                better = cost < bv
                bv = jnp.where(better, cost, bv)
                bi = jnp.where(better, lin, bi)
                bg = jnp.where(better, gam, bg)
                return bv, bi, bg

            bv, bi, bg = lax.fori_loop(16 * u, N // 16, v_body,
                                       (bv, bi, bg), unroll=4)

    stage[0, pl.ds(0, 16)] = bv
    stage[1, pl.ds(0, 16)] = bg
    stagei[0, pl.ds(0, 16)] = bi
    pltpu.sync_copy(stage, cand_hbm.at[wid])
    pltpu.sync_copy(stagei, candi_hbm.at[wid])


_scan = functools.partial(
    pl.kernel,
    out_type=(jax.ShapeDtypeStruct((NW, 8, 128), jnp.float32),
              jax.ShapeDtypeStruct((NW, 8, 128), jnp.int32)),
    mesh=plsc.VectorSubcoreMesh(core_axis_name="c", subcore_axis_name="s"),
    compiler_params=pltpu.CompilerParams(use_tc_tiling_on_sc=True),
    scratch_types=[
        pltpu.VMEM((N,), jnp.float32),
        pltpu.VMEM((2, 8, N), jnp.float32),
        pltpu.VMEM((8, 128), jnp.float32),
        pltpu.VMEM((8, 128), jnp.int32),
        pltpu.VMEM((16, 8, 128), jnp.float32),
        pltpu.VMEM((128,), jnp.float32),
        pltpu.VMEM_SHARED((N,), jnp.float32),
        pltpu.SemaphoreType.DMA,
        pltpu.SemaphoreType.DMA,
        pltpu.SemaphoreType.DMA,
    ],
)(_scan_body)


def _merge_body(cand_ref, candi_ref, out_ref):
    v = cand_ref[:, 0, :16]
    g = cand_ref[:, 1, :16]
    ix = candi_ref[:, 0, :16]
    m = jnp.min(v)
    win = jnp.min(jnp.where(v == m, ix, BIG_I32))
    gw = jnp.min(jnp.where(ix == win, g, jnp.inf))
    i_min = win >> 11
    j_min = win & (N - 1)
    flat = (lax.broadcasted_iota(jnp.int32, (16, 128), 0) * 128
            + lax.broadcasted_iota(jnp.int32, (16, 128), 1))
    out_ref[...] = (jnp.where(flat == i_min, gw, 0.0)
                    + jnp.where(flat == j_min, 1.0 - gw, 0.0))


_merge = pl.pallas_call(
    _merge_body,
    out_shape=jax.ShapeDtypeStruct((16, 128), jnp.float32),
)


def kernel(grammian):
    cand, candi = _scan(grammian)
    return _merge(cand, candi).reshape(N)


# speedup vs baseline: 12.0272x; 1.0979x over previous
import functools

import jax
import jax.numpy as jnp
from jax import lax
from jax.experimental import pallas as pl
from jax.experimental.pallas import tpu as pltpu
from jax.experimental.pallas import tpu_sc as plsc

N = 2048
NW = 32
BANDS_PER_W = 8
BIG_I32 = 2 ** 30


def _scan_body(g_hbm, cand_hbm, candi_hbm,
               ddiag_v, rowbuf, stage, stagei, dtiles, dloc, spdiag,
               rsem0, rsem1, dtsem):
    cid = lax.axis_index("c")
    sid = lax.axis_index("s")
    wid = sid * 2 + cid

    lane = lax.iota(jnp.int32, 16)
    rsems = (rsem0, rsem1)

    def start_band(u, buf):
        row0 = pl.multiple_of(8 * (wid + NW * u), 8)
        c0 = 256 * u
        return pltpu.async_copy(
            g_hbm.at[pl.ds(row0, 8), pl.ds(c0, N - c0)],
            rowbuf.at[buf, slice(None), pl.ds(c0, N - c0)],
            rsems[buf])

    band0_desc = start_band(0, 0)

    col0 = pl.multiple_of(128 * sid, 128)
    dt_descs = []
    for tt in range(16):
        row_off = pl.multiple_of(128 * sid + 8 * tt, 8)
        dt_descs.append(pltpu.async_copy(
            g_hbm.at[pl.ds(row_off, 8), pl.ds(col0, 128)],
            dtiles.at[tt], dtsem))
    for d in dt_descs:
        d.wait()
    for k in range(8):
        acc = jnp.zeros((16,), jnp.float32)
        for e in range(16):
            eidx = 16 * k + e
            tt, r = eidx // 8, eidx % 8
            c = 8 * tt + r
            chunk = dtiles[tt, r, pl.ds((c // 16) * 16, 16)]
            acc = jnp.where(lane == e, chunk[c % 16], acc)
        dloc[pl.ds(16 * k, 16)] = acc
    pltpu.sync_copy(dloc, spdiag.at[pl.ds(128 * sid, 128)])
    plsc.subcore_barrier()
    pltpu.sync_copy(spdiag, ddiag_v)

    w_even = (wid & 1) == 0
    bvs = tuple(jnp.full((16,), jnp.inf, jnp.float32) for _ in range(8))
    bis = tuple(jnp.full((16,), BIG_I32, jnp.int32) for _ in range(8))

    descs = [band0_desc, None]
    for u in range(BANDS_PER_W):
        buf = u % 2
        if u + 1 < BANDS_PER_W:
            descs[1 - buf] = start_band(u + 1, 1 - buf)
        descs[buf].wait()
        i0 = 8 * (wid + NW * u)
        dchunk = ddiag_v[pl.ds((i0 >> 4) * 16, 16)]
        a16s = []
        for r in range(8):
            a_s = jnp.where(w_even, dchunk[r], dchunk[r + 8])
            a16s.append(jnp.full((16,), a_s, jnp.float32))

        def v_body(v, carry, buf=buf, i0=i0, a16s=a16s):
            bvs, bis = carry
            b16 = ddiag_v[pl.ds(v * 16, 16)]
            j16 = v * 16 + lane
            nbvs, nbis = [], []
            for r in range(8):
                i = i0 + r
                a16 = a16s[r]
                c16 = rowbuf[buf, r, pl.ds(v * 16, 16)]
                t1 = b16 - c16
                den = a16 + b16 - 2.0 * c16 + 1e-8
                gam = t1 / den
                cr = b16 + gam * (c16 - b16)
                cost = jnp.where(c16 < b16, cr, b16)
                cost = jnp.where(c16 < a16, cost, a16)
                better = (cost < bvs[r]) & (j16 > i)
                nbvs.append(jnp.where(better, cost, bvs[r]))
                nbis.append(jnp.where(better, i * N + j16, bis[r]))
            return tuple(nbvs), tuple(nbis)

        bvs, bis = lax.fori_loop(16 * u, N // 16, v_body, (bvs, bis))

    for r in range(8):
        stage[r, pl.ds(0, 16)] = bvs[r]
        stagei[r, pl.ds(0, 16)] = bis[r]
    pltpu.sync_copy(stage, cand_hbm.at[wid])
    pltpu.sync_copy(stagei, candi_hbm.at[wid])


_scan = functools.partial(
    pl.kernel,
    out_type=(jax.ShapeDtypeStruct((NW, 8, 128), jnp.float32),
              jax.ShapeDtypeStruct((NW, 8, 128), jnp.int32)),
    mesh=plsc.VectorSubcoreMesh(core_axis_name="c", subcore_axis_name="s"),
    compiler_params=pltpu.CompilerParams(use_tc_tiling_on_sc=True),
    scratch_types=[
        pltpu.VMEM((N,), jnp.float32),
        pltpu.VMEM((2, 8, N), jnp.float32),
        pltpu.VMEM((8, 128), jnp.float32),
        pltpu.VMEM((8, 128), jnp.int32),
        pltpu.VMEM((16, 8, 128), jnp.float32),
        pltpu.VMEM((128,), jnp.float32),
        pltpu.VMEM_SHARED((N,), jnp.float32),
        pltpu.SemaphoreType.DMA,
        pltpu.SemaphoreType.DMA,
        pltpu.SemaphoreType.DMA,
    ],
)(_scan_body)


def _merge_body(cand_ref, candi_ref, g_ref, out_ref, cbuf, abuf, bbuf, msem):
    v = cand_ref[:, :, :16]
    ix = candi_ref[:, :, :16]
    m = jnp.min(v)
    win = jnp.min(jnp.where(v == m, ix, BIG_I32))
    i_min = win >> 11
    j_min = win & (N - 1)
    ja = pl.multiple_of((j_min >> 7) * 128, 128)
    ia = pl.multiple_of((i_min >> 7) * 128, 128)
    cp = pltpu.make_async_copy(
        g_ref.at[pl.ds(i_min, 1), pl.ds(ja, 128)], cbuf, msem)
    cp.start()
    cp.wait()
    ap = pltpu.make_async_copy(
        g_ref.at[pl.ds(i_min, 1), pl.ds(ia, 128)], abuf, msem)
    ap.start()
    ap.wait()
    bp = pltpu.make_async_copy(
        g_ref.at[pl.ds(j_min, 1), pl.ds(ja, 128)], bbuf, msem)
    bp.start()
    bp.wait()
    l8 = lax.broadcasted_iota(jnp.int32, (1, 128), 1)
    c = jnp.sum(jnp.where(l8 == (j_min & 127), cbuf[...], 0.0))
    a = jnp.sum(jnp.where(l8 == (i_min & 127), abuf[...], 0.0))
    b = jnp.sum(jnp.where(l8 == (j_min & 127), bbuf[...], 0.0))
    gw = (b - c) / (a + b - 2.0 * c + 1e-8)
    gw = jnp.where(c < b, gw, 0.0)
    gw = jnp.where(c < a, gw, 1.0)
    flat = (lax.broadcasted_iota(jnp.int32, (16, 128), 0) * 128
            + lax.broadcasted_iota(jnp.int32, (16, 128), 1))
    out_ref[...] = (jnp.where(flat == i_min, gw, 0.0)
                    + jnp.where(flat == j_min, 1.0 - gw, 0.0))


_merge = pl.pallas_call(
    _merge_body,
    in_specs=[
        pl.BlockSpec((NW, 8, 128), lambda: (0, 0, 0)),
        pl.BlockSpec((NW, 8, 128), lambda: (0, 0, 0)),
        pl.BlockSpec(memory_space=pl.ANY),
    ],
    out_shape=jax.ShapeDtypeStruct((16, 128), jnp.float32),
    scratch_shapes=[
        pltpu.VMEM((1, 128), jnp.float32),
        pltpu.VMEM((1, 128), jnp.float32),
        pltpu.VMEM((1, 128), jnp.float32),
        pltpu.SemaphoreType.DMA,
    ],
)


def kernel(grammian):
    cand, candi = _scan(grammian)
    return _merge(cand, candi, grammian).reshape(N)
